# Initial kernel scaffold; baseline (speedup 1.0000x reference)
import jax
import jax.numpy as jnp
from jax.experimental import pallas as pl


def kernel(h, w, row_weight, col_weight):
    raise NotImplementedError("write your pallas kernel here")


# speedup vs baseline: 3.0161x; 3.0161x over previous
import jax
import jax.numpy as jnp
from jax.experimental import pallas as pl


def kernel(h, w, row_weight, col_weight):
    H, HALF = row_weight.shape
    W = col_weight.shape[0]
    BI = 8

    def body(row_ref, col_ref, out_ref):
        row = row_ref[...]
        col = col_ref[...]
        out_ref[:, :, :HALF] = jnp.broadcast_to(
            row[:, None, :], (BI, W, HALF))
        out_ref[:, :, HALF:] = jnp.broadcast_to(
            col[None, :, :], (BI, W, HALF))

    out3 = pl.pallas_call(
        body,
        grid=(H // BI,),
        in_specs=[
            pl.BlockSpec((BI, HALF), lambda i: (i, 0)),
            pl.BlockSpec((W, HALF), lambda i: (0, 0)),
        ],
        out_specs=pl.BlockSpec((BI, W, 2 * HALF), lambda i: (i, 0, 0)),
        out_shape=jax.ShapeDtypeStruct((H, W, 2 * HALF), jnp.float32),
    )(row_weight, col_weight)
    return out3.reshape(H * W, 2 * HALF)
